# baseline (speedup 1.0000x reference)
import functools
import jax
import jax.numpy as jnp
from jax import lax
from jax.experimental import pallas as pl
from jax.experimental.pallas import tpu as pltpu
from jax.experimental.pallas import tpu_sc as plsc

N = 10000
E = 320000
D = 128
R = 4
EPS = 1e-10

NC = 2
NS = 16
SEG = N * R
SEG_HALF = SEG // NC
ACC_ROWS = SEG_HALF + 96
STRIPE = ACC_ROWS // NS
NFL = 10
FSTRIPE = SEG_HALF // NFL
CH = D // 2

EPP = E // (NC * NS)
PB = 80
PNB = EPP // PB
CBUF = EPP + 16
SB = 128
NSLOT = 3


def _prep_body(srcr, dstr, etr, bsrc, bsid, bcnt,
               ca_src, ca_sid, cb_src, cb_sid, cv0, cv1,
               ps0, pd0, pe0, ps1, pd1, pe1, semi0, semi1):
    c = lax.axis_index("c")
    s = lax.axis_index("s")
    w = c * NS + s
    ebase = w * EPP
    slots = ((ps0, pd0, pe0, semi0), (ps1, pd1, pe1, semi1))

    def issue(slot, i):
        off = jnp.minimum(ebase + i * PB, E - PB)
        pltpu.async_copy(srcr.at[pl.ds(off, PB)], slot[0], slot[3])
        pltpu.async_copy(dstr.at[pl.ds(off, PB)], slot[1], slot[3])
        pltpu.async_copy(etr.at[pl.ds(off, PB)], slot[2], slot[3])

    def waiti(slot):
        for buf in slot[:3]:
            pltpu.make_async_copy(srcr.at[pl.ds(0, PB)], buf, slot[3]).wait()

    issue(slots[0], 0)
    issue(slots[1], 1)

    def do_batch(slot, carry):
        def lane(j, cr):
            cA, cB = cr
            dsj = pl.ds(j * 16, 16)
            sv = slot[0][dsj]
            dv = slot[1][dsj]
            ev = slot[2][dsj]
            si = dv * R + ev
            okA = si < SEG_HALF
            incl = plsc.cumsum(okA.astype(jnp.int32))
            lanev = lax.iota(jnp.int32, 16)
            posA = jnp.where(okA, cA + incl - 1, EPP + lanev)
            posB = jnp.where(okA, EPP + lanev, cB + lanev - incl)
            plsc.store_scatter(ca_src, [posA], sv)
            plsc.store_scatter(ca_sid, [posA], si)
            plsc.store_scatter(cb_src, [posB], sv)
            plsc.store_scatter(cb_sid, [posB], si - SEG_HALF)
            d = jnp.sum(okA.astype(jnp.int32))
            return (cA + d, cB + (16 - d))

        return lax.fori_loop(0, PB // 16, lane, carry)

    def step(jb, carry):
        for b in range(2):
            i = 2 * jb + b
            slot = slots[b]
            waiti(slot)
            carry = do_batch(slot, carry)
            issue(slot, i + 2)
        return carry

    carry = lax.fori_loop(0, (PNB - 1) // 2, step, (0, 0))
    waiti(slots[0])
    cA, cB = do_batch(slots[0], carry)
    waiti(slots[1])
    pltpu.sync_copy(ca_src.at[pl.ds(0, EPP)], bsrc.at[pl.ds(ebase, EPP)])
    pltpu.sync_copy(ca_sid.at[pl.ds(0, EPP)], bsid.at[pl.ds(ebase, EPP)])
    pltpu.sync_copy(cb_src.at[pl.ds(0, EPP)], bsrc.at[pl.ds(E + ebase, EPP)])
    pltpu.sync_copy(cb_sid.at[pl.ds(0, EPP)], bsid.at[pl.ds(E + ebase, EPP)])
    cv0[pl.ds(0, 16)] = jnp.full((16,), cA, jnp.int32)
    cv1[pl.ds(0, 16)] = jnp.full((16,), cB, jnp.int32)
    pltpu.sync_copy(cv0, bcnt.at[pl.ds(w * 16, 16)])
    pltpu.sync_copy(cv1, bcnt.at[pl.ds((NC * NS + w) * 16, 16)])


@functools.lru_cache(maxsize=None)
def _make_prep():
    mesh = plsc.VectorSubcoreMesh(core_axis_name="c", subcore_axis_name="s",
                                  num_cores=NC, num_subcores=NS)
    scratch = [
        pltpu.VMEM((CBUF,), jnp.int32),
        pltpu.VMEM((CBUF,), jnp.int32),
        pltpu.VMEM((CBUF,), jnp.int32),
        pltpu.VMEM((CBUF,), jnp.int32),
        pltpu.VMEM((16,), jnp.int32),
        pltpu.VMEM((16,), jnp.int32),
        pltpu.VMEM((PB,), jnp.int32),
        pltpu.VMEM((PB,), jnp.int32),
        pltpu.VMEM((PB,), jnp.int32),
        pltpu.VMEM((PB,), jnp.int32),
        pltpu.VMEM((PB,), jnp.int32),
        pltpu.VMEM((PB,), jnp.int32),
        pltpu.SemaphoreType.DMA,
        pltpu.SemaphoreType.DMA,
    ]
    return pl.kernel(
        _prep_body,
        out_type=(jax.ShapeDtypeStruct((2 * E,), jnp.int32),
                  jax.ShapeDtypeStruct((2 * E,), jnp.int32),
                  jax.ShapeDtypeStruct((2 * NC * NS * 16,), jnp.int32)),
        mesh=mesh,
        scratch_types=tuple(scratch),
        compiler_params=pltpu.CompilerParams(use_tc_tiling_on_sc=False,
                                             needs_layout_passes=False),
    )


def _seg_body(with_counts, hv, bsrc, bsid, bcnt, zrows, znc, onesh, *rest):
    if with_counts:
        (s_out0, s_out1, cnt_out, acc, cnta, cbuf) = rest[:6]
        slot_refs = rest[6:6 + 5 * NSLOT]
        onesb = rest[6 + 5 * NSLOT]
        sems = rest[7 + 5 * NSLOT:]
    else:
        (s_out0, s_out1, acc, cbuf) = rest[:4]
        slot_refs = rest[4:4 + 5 * NSLOT]
        sems = rest[4 + 5 * NSLOT:]
        cnt_out = cnta = onesb = None
    s_outs = (s_out0, s_out1)
    slots = tuple(
        tuple(slot_refs[5 * b:5 * b + 5]) + (sems[2 * b], sems[2 * b + 1])
        for b in range(NSLOT))

    c = lax.axis_index("c")
    s = lax.axis_index("s")
    base = c * SEG_HALF

    pltpu.sync_copy(bcnt, cbuf)
    if with_counts:
        pltpu.sync_copy(onesh, onesb)

    def issue_idx(slot, ebase, i):
        e0 = jnp.minimum(ebase + i * SB, 2 * E - SB)
        pltpu.async_copy(bsrc.at[pl.ds(e0, SB)], slot[0], slot[5])
        pltpu.async_copy(bsid.at[pl.ds(e0, SB)], slot[1], slot[5])

    def wait_idx(slot):
        pltpu.make_async_copy(bsrc.at[pl.ds(0, SB)], slot[0], slot[5]).wait()
        pltpu.make_async_copy(bsrc.at[pl.ds(0, SB)], slot[1], slot[5]).wait()

    def compute(slot, p, i, cnt):
        srcb, sidb, gidb, sixb = slot[:4]

        def lane(j, _):
            dsj = pl.ds(j * 16, 16)
            kv = lax.iota(jnp.int32, 16) + (i * SB + j * 16)
            valid = kv < cnt
            sv = srcb[dsj]
            si = sidb[dsj]
            gidb[dsj] = jnp.where(valid, sv * 2 + p, 0)
            sixb[dsj] = jnp.where(valid, si, SEG_HALF)
            return 0

        lax.fori_loop(0, SB // 16, lane, 0)

    def issue_gather(slot):
        pltpu.async_copy(hv.at[slot[2]], slot[4], slot[6])

    def finish_scatter(slot, p):
        sixb, rowsb, semg = slot[3], slot[4], slot[6]
        pltpu.make_async_copy(hv.at[pl.ds(0, SB)], rowsb, semg).wait()
        pltpu.sync_copy(rowsb, acc.at[sixb], add=True)
        if with_counts and p == 0:
            pltpu.sync_copy(onesb, cnta.at[sixb], add=True)

    for p in range(2):
        pltpu.sync_copy(zrows, acc.at[pl.ds(s * STRIPE, STRIPE), :])
        if with_counts and p == 0:
            pltpu.sync_copy(znc, cnta.at[pl.ds(s * STRIPE, STRIPE), :])
        plsc.subcore_barrier()

        for k in range(2):
            w = 2 * s + k
            cnt = cbuf[pl.ds((c * NC * NS + w) * 16, 16)][0]
            ebase = c * E + w * EPP
            nbs = jnp.maximum((cnt + NSLOT * SB - 1) // (NSLOT * SB), 1)

            for b in range(NSLOT):
                issue_idx(slots[b], ebase, b)

            def step(jb, _):
                for b in range(NSLOT):
                    i = NSLOT * jb + b
                    slot = slots[b]
                    wait_idx(slot)
                    compute(slot, p, i, cnt)
                    issue_gather(slot)
                    issue_idx(slot, ebase, i + NSLOT)

                    @pl.when(i >= NSLOT - 1)
                    def _():
                        finish_scatter(slots[(b + 1) % NSLOT], p)

                return 0

            lax.fori_loop(0, nbs, step, 0)
            for b in range(1, NSLOT):
                finish_scatter(slots[b], p)
            for b in range(NSLOT):
                wait_idx(slots[b])
        plsc.subcore_barrier()

        @pl.when(s < NFL)
        def _flush():
            r0 = s * FSTRIPE
            pltpu.sync_copy(
                acc.at[pl.ds(r0, FSTRIPE), :],
                s_outs[p].at[pl.ds(base + r0, FSTRIPE), :])
            if with_counts and p == 0:
                pltpu.sync_copy(cnta.at[pl.ds(r0, FSTRIPE), :],
                                cnt_out.at[pl.ds(base + r0, FSTRIPE), :])

        plsc.subcore_barrier()


@functools.lru_cache(maxsize=None)
def _make_seg(with_counts):
    mesh = plsc.VectorSubcoreMesh(core_axis_name="c", subcore_axis_name="s",
                                  num_cores=NC, num_subcores=NS)
    out_type = [jax.ShapeDtypeStruct((SEG, CH), jnp.float32),
                jax.ShapeDtypeStruct((SEG, CH), jnp.float32)]
    scratch = [
        pltpu.VMEM_SHARED((ACC_ROWS, CH), jnp.float32),
    ]
    if with_counts:
        out_type.append(jax.ShapeDtypeStruct((SEG, 16), jnp.float32))
        scratch.append(pltpu.VMEM_SHARED((ACC_ROWS, 16), jnp.float32))
    scratch.append(pltpu.VMEM((2 * NC * NS * 16,), jnp.int32))
    for _slot in range(NSLOT):
        scratch += [
            pltpu.VMEM((SB,), jnp.int32),
            pltpu.VMEM((SB,), jnp.int32),
            pltpu.VMEM((SB,), jnp.int32),
            pltpu.VMEM((SB,), jnp.int32),
            pltpu.VMEM((SB, CH), jnp.float32),
        ]
    if with_counts:
        scratch.append(pltpu.VMEM((SB, 16), jnp.float32))
    scratch += [pltpu.SemaphoreType.DMA] * (2 * NSLOT)
    return pl.kernel(
        functools.partial(_seg_body, with_counts),
        out_type=tuple(out_type),
        mesh=mesh,
        scratch_types=tuple(scratch),
        compiler_params=pltpu.CompilerParams(use_tc_tiling_on_sc=False),
    )


def _dense_body(xin, prev, s, cnt, wl, wr, wpa, wpb, wta, wtb,
                b1, bp, bt, h_out, g_out):
    inv = 1.0 / (cnt[...] + EPS)
    acc = jnp.dot(xin[...], wl[...], preferred_element_type=jnp.float32)
    for r in range(R):
        upd = s[:, r * D:(r + 1) * D] * inv[:, r:r + 1]
        acc = acc + jnp.dot(upd, wr[r * D:(r + 1) * D, :],
                            preferred_element_type=jnp.float32)
    h = jax.nn.sigmoid(acc + b1[...])
    pv = prev[...]
    pa = jax.nn.relu(
        jnp.dot(h, wpa[...], preferred_element_type=jnp.float32)
        + jnp.dot(pv, wpb[...], preferred_element_type=jnp.float32)
        + bp[...])
    ga = jax.nn.sigmoid(
        jnp.dot(h, wta[...], preferred_element_type=jnp.float32)
        + jnp.dot(pv, wtb[...], preferred_element_type=jnp.float32)
        + bt[...])
    h_out[...] = h
    g_out[...] = ga * pa + (1.0 - ga) * h


_BN = 1000


def _dense_call(xin, prev, s, cnt4, wlT, wrT, wpaT, wpbT, wtaT, wtbT,
                b1, bp, bt):
    grid = (N // _BN,)
    row = lambda i: (i, 0)
    const = lambda i: (0, 0)
    return pl.pallas_call(
        _dense_body,
        grid=grid,
        in_specs=[
            pl.BlockSpec((_BN, D), row),
            pl.BlockSpec((_BN, D), row),
            pl.BlockSpec((_BN, R * D), row),
            pl.BlockSpec((_BN, R), row),
            pl.BlockSpec((D, D), const),
            pl.BlockSpec((R * D, D), const),
            pl.BlockSpec((D, D), const),
            pl.BlockSpec((D, D), const),
            pl.BlockSpec((D, D), const),
            pl.BlockSpec((D, D), const),
            pl.BlockSpec((1, D), const),
            pl.BlockSpec((1, D), const),
            pl.BlockSpec((1, D), const),
        ],
        out_specs=[
            pl.BlockSpec((_BN, D), row),
            pl.BlockSpec((_BN, D), row),
        ],
        out_shape=[
            jax.ShapeDtypeStruct((N, D), jnp.float32),
            jax.ShapeDtypeStruct((N, D), jnp.float32),
        ],
    )(xin, prev, s, cnt4, wlT, wrT, wpaT, wpbT, wtaT, wtbT, b1, bp, bt)


def kernel(x, edge_index, edge_type,
           Wl1, bl1, Wr1, br1, Wp1, bp1, Wt1, bt1,
           Wl2, bl2, Wr2, br2, Wp2, bp2, Wt2, bt2):
    src = edge_index[0]
    dst = edge_index[1]

    zrows = jnp.zeros((STRIPE, CH), jnp.float32)
    znc = jnp.zeros((STRIPE, 16), jnp.float32)
    onesh = jnp.zeros((SB, 16), jnp.float32).at[:, 0].set(1.0)

    bsrc, bsid, bcnt = _make_prep()(src, dst, edge_type)

    s1a, s1b, cnt = _make_seg(True)(x.reshape(2 * N, CH), bsrc, bsid, bcnt,
                                    zrows, znc, onesh)
    s1 = jnp.concatenate([s1a, s1b], axis=-1)
    cnt4 = cnt[:, 0].reshape(N, R)
    h1, g1 = _dense_call(
        x, x, s1.reshape(N, R * D), cnt4,
        Wl1.T, Wr1.T, Wp1[:, :D].T, Wp1[:, D:].T, Wt1[:, :D].T, Wt1[:, D:].T,
        (bl1 + br1).reshape(1, D), bp1.reshape(1, D), bt1.reshape(1, D))

    s2a, s2b = _make_seg(False)(g1.reshape(2 * N, CH), bsrc, bsid, bcnt,
                                zrows, znc, onesh)
    s2 = jnp.concatenate([s2a, s2b], axis=-1)
    _, g2 = _dense_call(
        g1, h1, s2.reshape(N, R * D), cnt4,
        Wl2.T, Wr2.T, Wp2[:, :D].T, Wp2[:, D:].T, Wt2[:, :D].T, Wt2[:, D:].T,
        (bl2 + br2).reshape(1, D), bp2.reshape(1, D), bt2.reshape(1, D))
    return g2


# speedup vs baseline: 1.2289x; 1.2289x over previous
import functools
import jax
import jax.numpy as jnp
from jax import lax
from jax.experimental import pallas as pl
from jax.experimental.pallas import tpu as pltpu
from jax.experimental.pallas import tpu_sc as plsc

N = 10000
E = 320000
D = 128
R = 4
EPS = 1e-10

NC = 2
NS = 16
NQ = 4
SEG = N * R
SEG_Q = SEG // NQ
ACC_ROWS = SEG_Q + 112
STRIPE = ACC_ROWS // NS
NFL = 10
FSTRIPE = SEG_Q // NFL

EPP = E // (NC * NS)
PB = 80
PNB = EPP // PB
CBUF = EPP + 16
SB = 128


def _prep_body(srcr, dstr, etr, bsrc, bsid, bcnt, *rest):
    csrc = rest[0:NQ]
    csid = rest[NQ:2 * NQ]
    cv = rest[2 * NQ:3 * NQ]
    (ps0, pd0, pe0, ps1, pd1, pe1, semi0, semi1) = rest[3 * NQ:]
    c = lax.axis_index("c")
    s = lax.axis_index("s")
    w = c * NS + s
    ebase = w * EPP
    slots = ((ps0, pd0, pe0, semi0), (ps1, pd1, pe1, semi1))

    def issue(slot, i):
        off = jnp.minimum(ebase + i * PB, E - PB)
        pltpu.async_copy(srcr.at[pl.ds(off, PB)], slot[0], slot[3])
        pltpu.async_copy(dstr.at[pl.ds(off, PB)], slot[1], slot[3])
        pltpu.async_copy(etr.at[pl.ds(off, PB)], slot[2], slot[3])

    def waiti(slot):
        for buf in slot[:3]:
            pltpu.make_async_copy(srcr.at[pl.ds(0, PB)], buf, slot[3]).wait()

    issue(slots[0], 0)
    issue(slots[1], 1)

    def do_batch(slot, carry):
        def lane(j, cr):
            dsj = pl.ds(j * 16, 16)
            sv = slot[0][dsj]
            dv = slot[1][dsj]
            ev = slot[2][dsj]
            si = dv * R + ev
            lanev = lax.iota(jnp.int32, 16)
            out = []
            for q in range(NQ):
                okq = (si >= q * SEG_Q) & (si < (q + 1) * SEG_Q)
                incl = plsc.cumsum(okq.astype(jnp.int32))
                posq = jnp.where(okq, cr[q] + incl - 1, EPP + lanev)
                plsc.store_scatter(csrc[q], [posq], sv)
                plsc.store_scatter(csid[q], [posq], si - q * SEG_Q)
                out.append(cr[q] + jnp.sum(okq.astype(jnp.int32)))
            return tuple(out)

        return lax.fori_loop(0, PB // 16, lane, carry)

    def step(jb, carry):
        for b in range(2):
            i = 2 * jb + b
            slot = slots[b]
            waiti(slot)
            carry = do_batch(slot, carry)
            issue(slot, i + 2)
        return carry

    carry = lax.fori_loop(0, (PNB - 1) // 2, step, (0,) * NQ)
    waiti(slots[0])
    carry = do_batch(slots[0], carry)
    waiti(slots[1])
    for q in range(NQ):
        pltpu.sync_copy(csrc[q].at[pl.ds(0, EPP)],
                        bsrc.at[pl.ds(q * E + ebase, EPP)])
        pltpu.sync_copy(csid[q].at[pl.ds(0, EPP)],
                        bsid.at[pl.ds(q * E + ebase, EPP)])
        cv[q][pl.ds(0, 16)] = jnp.full((16,), carry[q], jnp.int32)
        pltpu.sync_copy(cv[q], bcnt.at[pl.ds((q * NC * NS + w) * 16, 16)])


@functools.lru_cache(maxsize=None)
def _make_prep():
    mesh = plsc.VectorSubcoreMesh(core_axis_name="c", subcore_axis_name="s",
                                  num_cores=NC, num_subcores=NS)
    scratch = (
        [pltpu.VMEM((CBUF,), jnp.int32)] * NQ
        + [pltpu.VMEM((CBUF,), jnp.int32)] * NQ
        + [pltpu.VMEM((16,), jnp.int32)] * NQ
        + [pltpu.VMEM((PB,), jnp.int32)] * 6
        + [pltpu.SemaphoreType.DMA] * 2
    )
    return pl.kernel(
        _prep_body,
        out_type=(jax.ShapeDtypeStruct((NQ * E,), jnp.int32),
                  jax.ShapeDtypeStruct((NQ * E,), jnp.int32),
                  jax.ShapeDtypeStruct((NQ * NC * NS * 16,), jnp.int32)),
        mesh=mesh,
        scratch_types=tuple(scratch),
        compiler_params=pltpu.CompilerParams(use_tc_tiling_on_sc=False,
                                             needs_layout_passes=False),
    )


def _seg_body(with_counts, hv, bsrc, bsid, bcnt, zrows, znc, onesh, *rest):
    if with_counts:
        (s_out, cnt_out, acc, cnta, cbuf,
         src0, sid0, gid0, six0, rows0,
         src1, sid1, gid1, six1, rows1,
         onesb, semi0, semi1, semg0, semg1) = rest
    else:
        (s_out, acc, cbuf,
         src0, sid0, gid0, six0, rows0,
         src1, sid1, gid1, six1, rows1,
         semi0, semi1, semg0, semg1) = rest
        cnt_out = cnta = onesb = None
    slots = ((src0, sid0, gid0, six0, rows0, semi0, semg0),
             (src1, sid1, gid1, six1, rows1, semi1, semg1))

    c = lax.axis_index("c")
    s = lax.axis_index("s")

    pltpu.sync_copy(bcnt, cbuf)
    if with_counts:
        pltpu.sync_copy(onesh, onesb)

    def issue_idx(slot, ebase, i):
        e0 = jnp.minimum(ebase + i * SB, NQ * E - SB)
        pltpu.async_copy(bsrc.at[pl.ds(e0, SB)], slot[0], slot[5])
        pltpu.async_copy(bsid.at[pl.ds(e0, SB)], slot[1], slot[5])

    def wait_idx(slot):
        pltpu.make_async_copy(bsrc.at[pl.ds(0, SB)], slot[0], slot[5]).wait()
        pltpu.make_async_copy(bsrc.at[pl.ds(0, SB)], slot[1], slot[5]).wait()

    def compute(slot, i, cnt):
        srcb, sidb, gidb, sixb = slot[:4]

        def lane(j, _):
            dsj = pl.ds(j * 16, 16)
            kv = lax.iota(jnp.int32, 16) + (i * SB + j * 16)
            valid = kv < cnt
            sv = srcb[dsj]
            si = sidb[dsj]
            gidb[dsj] = jnp.where(valid, sv, 0)
            sixb[dsj] = jnp.where(valid, si, SEG_Q)
            return 0

        lax.fori_loop(0, SB // 16, lane, 0)

    def issue_gather(slot):
        pltpu.async_copy(hv.at[slot[2]], slot[4], slot[6])

    def finish_scatter(slot):
        sixb, rowsb, semg = slot[3], slot[4], slot[6]
        pltpu.make_async_copy(hv.at[pl.ds(0, SB)], rowsb, semg).wait()
        pltpu.sync_copy(rowsb, acc.at[sixb], add=True)
        if with_counts:
            pltpu.sync_copy(onesb, cnta.at[sixb], add=True)

    for ph in range(2):
        pltpu.sync_copy(zrows, acc.at[pl.ds(s * STRIPE, STRIPE), :])
        if with_counts:
            pltpu.sync_copy(znc, cnta.at[pl.ds(s * STRIPE, STRIPE), :])
        plsc.subcore_barrier()
        q = 2 * c + ph

        for k in range(2):
            w = 2 * s + k
            cnt = cbuf[pl.ds((q * NC * NS + w) * 16, 16)][0]
            ebase = q * E + w * EPP
            nb2 = jnp.maximum((cnt + 2 * SB - 1) // (2 * SB), 1)

            issue_idx(slots[0], ebase, 0)
            issue_idx(slots[1], ebase, 1)

            def step(jb, _):
                for b in range(2):
                    i = 2 * jb + b
                    slot = slots[b]
                    wait_idx(slot)
                    compute(slot, i, cnt)
                    issue_gather(slot)
                    issue_idx(slot, ebase, i + 2)

                    @pl.when(i > 0)
                    def _():
                        finish_scatter(slots[1 - b])

                return 0

            lax.fori_loop(0, nb2, step, 0)
            finish_scatter(slots[1])
            wait_idx(slots[0])
            wait_idx(slots[1])
        plsc.subcore_barrier()

        @pl.when(s < NFL)
        def _flush():
            r0 = s * FSTRIPE
            pltpu.sync_copy(
                acc.at[pl.ds(r0, FSTRIPE), :],
                s_out.at[pl.ds(q * SEG_Q + r0, FSTRIPE), :])
            if with_counts:
                pltpu.sync_copy(cnta.at[pl.ds(r0, FSTRIPE), :],
                                cnt_out.at[pl.ds(q * SEG_Q + r0, FSTRIPE), :])

        plsc.subcore_barrier()


@functools.lru_cache(maxsize=None)
def _make_seg(with_counts):
    mesh = plsc.VectorSubcoreMesh(core_axis_name="c", subcore_axis_name="s",
                                  num_cores=NC, num_subcores=NS)
    out_type = [jax.ShapeDtypeStruct((SEG, D), jnp.float32)]
    scratch = [
        pltpu.VMEM_SHARED((ACC_ROWS, D), jnp.float32),
    ]
    if with_counts:
        out_type.append(jax.ShapeDtypeStruct((SEG, 16), jnp.float32))
        scratch.append(pltpu.VMEM_SHARED((ACC_ROWS, 16), jnp.float32))
    scratch.append(pltpu.VMEM((NQ * NC * NS * 16,), jnp.int32))
    for _slot in range(2):
        scratch += [
            pltpu.VMEM((SB,), jnp.int32),
            pltpu.VMEM((SB,), jnp.int32),
            pltpu.VMEM((SB,), jnp.int32),
            pltpu.VMEM((SB,), jnp.int32),
            pltpu.VMEM((SB, D), jnp.float32),
        ]
    if with_counts:
        scratch.append(pltpu.VMEM((SB, 16), jnp.float32))
    scratch += [pltpu.SemaphoreType.DMA] * 4
    return pl.kernel(
        functools.partial(_seg_body, with_counts),
        out_type=tuple(out_type),
        mesh=mesh,
        scratch_types=tuple(scratch),
        compiler_params=pltpu.CompilerParams(use_tc_tiling_on_sc=False),
    )


def _dense_body(xin, prev, s, cnt, wl, wr, wpa, wpb, wta, wtb,
                b1, bp, bt, h_out, g_out):
    inv = 1.0 / (cnt[...] + EPS)
    acc = jnp.dot(xin[...], wl[...], preferred_element_type=jnp.float32)
    for r in range(R):
        upd = s[:, r * D:(r + 1) * D] * inv[:, r:r + 1]
        acc = acc + jnp.dot(upd, wr[r * D:(r + 1) * D, :],
                            preferred_element_type=jnp.float32)
    h = jax.nn.sigmoid(acc + b1[...])
    pv = prev[...]
    pa = jax.nn.relu(
        jnp.dot(h, wpa[...], preferred_element_type=jnp.float32)
        + jnp.dot(pv, wpb[...], preferred_element_type=jnp.float32)
        + bp[...])
    ga = jax.nn.sigmoid(
        jnp.dot(h, wta[...], preferred_element_type=jnp.float32)
        + jnp.dot(pv, wtb[...], preferred_element_type=jnp.float32)
        + bt[...])
    h_out[...] = h
    g_out[...] = ga * pa + (1.0 - ga) * h


_BN = 1000


def _dense_call(xin, prev, s, cnt4, wlT, wrT, wpaT, wpbT, wtaT, wtbT,
                b1, bp, bt):
    grid = (N // _BN,)
    row = lambda i: (i, 0)
    const = lambda i: (0, 0)
    return pl.pallas_call(
        _dense_body,
        grid=grid,
        in_specs=[
            pl.BlockSpec((_BN, D), row),
            pl.BlockSpec((_BN, D), row),
            pl.BlockSpec((_BN, R * D), row),
            pl.BlockSpec((_BN, R), row),
            pl.BlockSpec((D, D), const),
            pl.BlockSpec((R * D, D), const),
            pl.BlockSpec((D, D), const),
            pl.BlockSpec((D, D), const),
            pl.BlockSpec((D, D), const),
            pl.BlockSpec((D, D), const),
            pl.BlockSpec((1, D), const),
            pl.BlockSpec((1, D), const),
            pl.BlockSpec((1, D), const),
        ],
        out_specs=[
            pl.BlockSpec((_BN, D), row),
            pl.BlockSpec((_BN, D), row),
        ],
        out_shape=[
            jax.ShapeDtypeStruct((N, D), jnp.float32),
            jax.ShapeDtypeStruct((N, D), jnp.float32),
        ],
    )(xin, prev, s, cnt4, wlT, wrT, wpaT, wpbT, wtaT, wtbT, b1, bp, bt)


def kernel(x, edge_index, edge_type,
           Wl1, bl1, Wr1, br1, Wp1, bp1, Wt1, bt1,
           Wl2, bl2, Wr2, br2, Wp2, bp2, Wt2, bt2):
    src = edge_index[0]
    dst = edge_index[1]

    zrows = jnp.zeros((STRIPE, D), jnp.float32)
    znc = jnp.zeros((STRIPE, 16), jnp.float32)
    onesh = jnp.zeros((SB, 16), jnp.float32).at[:, 0].set(1.0)

    bsrc, bsid, bcnt = _make_prep()(src, dst, edge_type)

    s1, cnt = _make_seg(True)(x, bsrc, bsid, bcnt, zrows, znc, onesh)
    cnt4 = cnt[:, 0].reshape(N, R)
    h1, g1 = _dense_call(
        x, x, s1.reshape(N, R * D), cnt4,
        Wl1.T, Wr1.T, Wp1[:, :D].T, Wp1[:, D:].T, Wt1[:, :D].T, Wt1[:, D:].T,
        (bl1 + br1).reshape(1, D), bp1.reshape(1, D), bt1.reshape(1, D))

    (s2,) = _make_seg(False)(g1, bsrc, bsid, bcnt, zrows, znc, onesh)
    _, g2 = _dense_call(
        g1, h1, s2.reshape(N, R * D), cnt4,
        Wl2.T, Wr2.T, Wp2[:, :D].T, Wp2[:, D:].T, Wt2[:, :D].T, Wt2[:, D:].T,
        (bl2 + br2).reshape(1, D), bp2.reshape(1, D), bt2.reshape(1, D))
    return g2


# speedup vs baseline: 1.3775x; 1.1210x over previous
import functools
import jax
import jax.numpy as jnp
from jax import lax
from jax.experimental import pallas as pl
from jax.experimental.pallas import tpu as pltpu
from jax.experimental.pallas import tpu_sc as plsc

N = 10000
E = 320000
D = 128
R = 4
EPS = 1e-10

NC = 2
NS = 16
SEG = N * R
SEG_HALF = SEG // NC
ACC_ROWS = SEG_HALF + 96
STRIPE = ACC_ROWS // NS
NFL = 10
FSTRIPE = SEG_HALF // NFL
CH = D // 2

EPP = E // (NC * NS)
PB = 80
PNB = EPP // PB
CBUF = EPP + 16
SB = 128


def _prep_body(srcr, dstr, etr, bsrc, bsid, bcnt,
               ca_src, ca_sid, cb_src, cb_sid, cv0, cv1,
               ps0, pd0, pe0, ps1, pd1, pe1, semi0, semi1):
    c = lax.axis_index("c")
    s = lax.axis_index("s")
    w = c * NS + s
    ebase = w * EPP
    slots = ((ps0, pd0, pe0, semi0), (ps1, pd1, pe1, semi1))

    def issue(slot, i):
        off = jnp.minimum(ebase + i * PB, E - PB)
        pltpu.async_copy(srcr.at[pl.ds(off, PB)], slot[0], slot[3])
        pltpu.async_copy(dstr.at[pl.ds(off, PB)], slot[1], slot[3])
        pltpu.async_copy(etr.at[pl.ds(off, PB)], slot[2], slot[3])

    def waiti(slot):
        for buf in slot[:3]:
            pltpu.make_async_copy(srcr.at[pl.ds(0, PB)], buf, slot[3]).wait()

    issue(slots[0], 0)
    issue(slots[1], 1)

    def do_batch(slot, carry):
        def lane(j, cr):
            cA, cB = cr
            dsj = pl.ds(j * 16, 16)
            sv = slot[0][dsj]
            dv = slot[1][dsj]
            ev = slot[2][dsj]
            si = dv * R + ev
            okA = si < SEG_HALF
            incl = plsc.cumsum(okA.astype(jnp.int32))
            lanev = lax.iota(jnp.int32, 16)
            posA = jnp.where(okA, cA + incl - 1, EPP + lanev)
            posB = jnp.where(okA, EPP + lanev, cB + lanev - incl)
            plsc.store_scatter(ca_src, [posA], sv)
            plsc.store_scatter(ca_sid, [posA], si)
            plsc.store_scatter(cb_src, [posB], sv)
            plsc.store_scatter(cb_sid, [posB], si - SEG_HALF)
            d = jnp.sum(okA.astype(jnp.int32))
            return (cA + d, cB + (16 - d))

        return lax.fori_loop(0, PB // 16, lane, carry)

    def step(jb, carry):
        for b in range(2):
            i = 2 * jb + b
            slot = slots[b]
            waiti(slot)
            carry = do_batch(slot, carry)
            issue(slot, i + 2)
        return carry

    carry = lax.fori_loop(0, (PNB - 1) // 2, step, (0, 0))
    waiti(slots[0])
    cA, cB = do_batch(slots[0], carry)
    waiti(slots[1])
    pltpu.sync_copy(ca_src.at[pl.ds(0, EPP)], bsrc.at[pl.ds(ebase, EPP)])
    pltpu.sync_copy(ca_sid.at[pl.ds(0, EPP)], bsid.at[pl.ds(ebase, EPP)])
    pltpu.sync_copy(cb_src.at[pl.ds(0, EPP)], bsrc.at[pl.ds(E + ebase, EPP)])
    pltpu.sync_copy(cb_sid.at[pl.ds(0, EPP)], bsid.at[pl.ds(E + ebase, EPP)])
    cv0[pl.ds(0, 16)] = jnp.full((16,), cA, jnp.int32)
    cv1[pl.ds(0, 16)] = jnp.full((16,), cB, jnp.int32)
    pltpu.sync_copy(cv0, bcnt.at[pl.ds(w * 16, 16)])
    pltpu.sync_copy(cv1, bcnt.at[pl.ds((NC * NS + w) * 16, 16)])


@functools.lru_cache(maxsize=None)
def _make_prep():
    mesh = plsc.VectorSubcoreMesh(core_axis_name="c", subcore_axis_name="s",
                                  num_cores=NC, num_subcores=NS)
    scratch = [
        pltpu.VMEM((CBUF,), jnp.int32),
        pltpu.VMEM((CBUF,), jnp.int32),
        pltpu.VMEM((CBUF,), jnp.int32),
        pltpu.VMEM((CBUF,), jnp.int32),
        pltpu.VMEM((16,), jnp.int32),
        pltpu.VMEM((16,), jnp.int32),
        pltpu.VMEM((PB,), jnp.int32),
        pltpu.VMEM((PB,), jnp.int32),
        pltpu.VMEM((PB,), jnp.int32),
        pltpu.VMEM((PB,), jnp.int32),
        pltpu.VMEM((PB,), jnp.int32),
        pltpu.VMEM((PB,), jnp.int32),
        pltpu.SemaphoreType.DMA,
        pltpu.SemaphoreType.DMA,
    ]
    return pl.kernel(
        _prep_body,
        out_type=(jax.ShapeDtypeStruct((2 * E,), jnp.int32),
                  jax.ShapeDtypeStruct((2 * E,), jnp.int32),
                  jax.ShapeDtypeStruct((2 * NC * NS * 16,), jnp.int32)),
        mesh=mesh,
        scratch_types=tuple(scratch),
        compiler_params=pltpu.CompilerParams(use_tc_tiling_on_sc=False,
                                             needs_layout_passes=False),
    )


def _seg_body(with_counts, hv, bsrc, bsid, bcnt, zrows, znc, onesh, *rest):
    if with_counts:
        (s_out0, s_out1, cnt_out, acc, cnta, cbuf,
         src0, sid0, gid0, six0, rows0,
         src1, sid1, gid1, six1, rows1,
         onesb, semi0, semi1, semg0, semg1) = rest
    else:
        (s_out0, s_out1, acc, cbuf,
         src0, sid0, gid0, six0, rows0,
         src1, sid1, gid1, six1, rows1,
         semi0, semi1, semg0, semg1) = rest
        cnt_out = cnta = onesb = None
    s_outs = (s_out0, s_out1)
    slots = ((src0, sid0, gid0, six0, rows0, semi0, semg0),
             (src1, sid1, gid1, six1, rows1, semi1, semg1))

    c = lax.axis_index("c")
    s = lax.axis_index("s")
    base = c * SEG_HALF

    pltpu.sync_copy(bcnt, cbuf)
    if with_counts:
        pltpu.sync_copy(onesh, onesb)

    def issue_idx(slot, ebase, i):
        e0 = jnp.minimum(ebase + i * SB, 2 * E - SB)
        pltpu.async_copy(bsrc.at[pl.ds(e0, SB)], slot[0], slot[5])
        pltpu.async_copy(bsid.at[pl.ds(e0, SB)], slot[1], slot[5])

    def wait_idx(slot):
        pltpu.make_async_copy(bsrc.at[pl.ds(0, SB)], slot[0], slot[5]).wait()
        pltpu.make_async_copy(bsrc.at[pl.ds(0, SB)], slot[1], slot[5]).wait()

    def compute(slot, p, i, cnt):
        srcb, sidb, gidb, sixb = slot[:4]

        def lane(j, _):
            dsj = pl.ds(j * 16, 16)
            kv = lax.iota(jnp.int32, 16) + (i * SB + j * 16)
            valid = kv < cnt
            sv = srcb[dsj]
            si = sidb[dsj]
            gidb[dsj] = jnp.where(valid, sv * 2 + p, 0)
            sixb[dsj] = jnp.where(valid, si, SEG_HALF)
            return 0

        lax.fori_loop(0, SB // 16, lane, 0)

    def issue_gather(slot):
        pltpu.async_copy(hv.at[slot[2]], slot[4], slot[6])

    def finish_scatter(slot, p):
        sixb, rowsb, semg = slot[3], slot[4], slot[6]
        pltpu.make_async_copy(hv.at[pl.ds(0, SB)], rowsb, semg).wait()
        pltpu.sync_copy(rowsb, acc.at[sixb], add=True)
        if with_counts and p == 0:
            pltpu.sync_copy(onesb, cnta.at[sixb], add=True)

    for p in range(2):
        pltpu.sync_copy(zrows, acc.at[pl.ds(s * STRIPE, STRIPE), :])
        if with_counts and p == 0:
            pltpu.sync_copy(znc, cnta.at[pl.ds(s * STRIPE, STRIPE), :])
        plsc.subcore_barrier()

        for k in range(2):
            w = 2 * s + k
            cnt = cbuf[pl.ds((c * NC * NS + w) * 16, 16)][0]
            ebase = c * E + w * EPP
            nb2 = jnp.maximum((cnt + 2 * SB - 1) // (2 * SB), 1)

            issue_idx(slots[0], ebase, 0)
            issue_idx(slots[1], ebase, 1)

            def step(jb, _):
                for b in range(2):
                    i = 2 * jb + b
                    slot = slots[b]
                    wait_idx(slot)
                    compute(slot, p, i, cnt)
                    issue_gather(slot)
                    issue_idx(slot, ebase, i + 2)

                    @pl.when(i > 0)
                    def _():
                        finish_scatter(slots[1 - b], p)

                return 0

            lax.fori_loop(0, nb2, step, 0)
            finish_scatter(slots[1], p)
            wait_idx(slots[0])
            wait_idx(slots[1])
        plsc.subcore_barrier()

        @pl.when(s < NFL)
        def _flush():
            r0 = s * FSTRIPE
            pltpu.sync_copy(
                acc.at[pl.ds(r0, FSTRIPE), :],
                s_outs[p].at[pl.ds(base + r0, FSTRIPE), :])
            if with_counts and p == 0:
                pltpu.sync_copy(cnta.at[pl.ds(r0, FSTRIPE), :],
                                cnt_out.at[pl.ds(base + r0, FSTRIPE), :])

        plsc.subcore_barrier()


@functools.lru_cache(maxsize=None)
def _make_seg(with_counts):
    mesh = plsc.VectorSubcoreMesh(core_axis_name="c", subcore_axis_name="s",
                                  num_cores=NC, num_subcores=NS)
    out_type = [jax.ShapeDtypeStruct((SEG, CH), jnp.float32),
                jax.ShapeDtypeStruct((SEG, CH), jnp.float32)]
    scratch = [
        pltpu.VMEM_SHARED((ACC_ROWS, CH), jnp.float32),
    ]
    if with_counts:
        out_type.append(jax.ShapeDtypeStruct((SEG, 16), jnp.float32))
        scratch.append(pltpu.VMEM_SHARED((ACC_ROWS, 16), jnp.float32))
    scratch.append(pltpu.VMEM((2 * NC * NS * 16,), jnp.int32))
    for _slot in range(2):
        scratch += [
            pltpu.VMEM((SB,), jnp.int32),
            pltpu.VMEM((SB,), jnp.int32),
            pltpu.VMEM((SB,), jnp.int32),
            pltpu.VMEM((SB,), jnp.int32),
            pltpu.VMEM((SB, CH), jnp.float32),
        ]
    if with_counts:
        scratch.append(pltpu.VMEM((SB, 16), jnp.float32))
    scratch += [pltpu.SemaphoreType.DMA] * 4
    return pl.kernel(
        functools.partial(_seg_body, with_counts),
        out_type=tuple(out_type),
        mesh=mesh,
        scratch_types=tuple(scratch),
        compiler_params=pltpu.CompilerParams(use_tc_tiling_on_sc=False),
    )


def _dense_body(xin, prev, s0, s1, cnt, wl, wra, wrb, wpa, wpb, wta, wtb,
                b1, bp, bt, h_out, g_out):
    inv = 1.0 / (cnt[...] + EPS)
    acc = jnp.dot(xin[...], wl[...], preferred_element_type=jnp.float32)
    for r in range(R):
        iv = inv[:, r:r + 1]
        u0 = s0[:, r * CH:(r + 1) * CH] * iv
        u1 = s1[:, r * CH:(r + 1) * CH] * iv
        acc = acc + jnp.dot(u0, wra[r * CH:(r + 1) * CH, :],
                            preferred_element_type=jnp.float32)
        acc = acc + jnp.dot(u1, wrb[r * CH:(r + 1) * CH, :],
                            preferred_element_type=jnp.float32)
    h = jax.nn.sigmoid(acc + b1[...])
    pv = prev[...]
    pa = jax.nn.relu(
        jnp.dot(h, wpa[...], preferred_element_type=jnp.float32)
        + jnp.dot(pv, wpb[...], preferred_element_type=jnp.float32)
        + bp[...])
    ga = jax.nn.sigmoid(
        jnp.dot(h, wta[...], preferred_element_type=jnp.float32)
        + jnp.dot(pv, wtb[...], preferred_element_type=jnp.float32)
        + bt[...])
    h_out[...] = h
    g_out[...] = ga * pa + (1.0 - ga) * h


_BN = 1000


def _dense_call(xin, prev, s0, s1, cnt4, wlT, wrA, wrB, wpaT, wpbT,
                wtaT, wtbT, b1, bp, bt):
    grid = (N // _BN,)
    row = lambda i: (i, 0)
    const = lambda i: (0, 0)
    return pl.pallas_call(
        _dense_body,
        grid=grid,
        in_specs=[
            pl.BlockSpec((_BN, D), row),
            pl.BlockSpec((_BN, D), row),
            pl.BlockSpec((_BN, R * CH), row),
            pl.BlockSpec((_BN, R * CH), row),
            pl.BlockSpec((_BN, R), row),
            pl.BlockSpec((D, D), const),
            pl.BlockSpec((R * CH, D), const),
            pl.BlockSpec((R * CH, D), const),
            pl.BlockSpec((D, D), const),
            pl.BlockSpec((D, D), const),
            pl.BlockSpec((D, D), const),
            pl.BlockSpec((D, D), const),
            pl.BlockSpec((1, D), const),
            pl.BlockSpec((1, D), const),
            pl.BlockSpec((1, D), const),
        ],
        out_specs=[
            pl.BlockSpec((_BN, D), row),
            pl.BlockSpec((_BN, D), row),
        ],
        out_shape=[
            jax.ShapeDtypeStruct((N, D), jnp.float32),
            jax.ShapeDtypeStruct((N, D), jnp.float32),
        ],
    )(xin, prev, s0, s1, cnt4, wlT, wrA, wrB, wpaT, wpbT, wtaT, wtbT,
      b1, bp, bt)


def kernel(x, edge_index, edge_type,
           Wl1, bl1, Wr1, br1, Wp1, bp1, Wt1, bt1,
           Wl2, bl2, Wr2, br2, Wp2, bp2, Wt2, bt2):
    src = edge_index[0]
    dst = edge_index[1]

    zrows = jnp.zeros((STRIPE, CH), jnp.float32)
    znc = jnp.zeros((STRIPE, 16), jnp.float32)
    onesh = jnp.zeros((SB, 16), jnp.float32).at[:, 0].set(1.0)

    bsrc, bsid, bcnt = _make_prep()(src, dst, edge_type)

    s1a, s1b, cnt = _make_seg(True)(x.reshape(2 * N, CH), bsrc, bsid, bcnt,
                                    zrows, znc, onesh)
    cnt4 = cnt[:, 0].reshape(N, R)
    wr1 = Wr1.T.reshape(R, 2, CH, D)
    h1, g1 = _dense_call(
        x, x, s1a.reshape(N, R * CH), s1b.reshape(N, R * CH), cnt4,
        Wl1.T, wr1[:, 0].reshape(R * CH, D), wr1[:, 1].reshape(R * CH, D),
        Wp1[:, :D].T, Wp1[:, D:].T, Wt1[:, :D].T, Wt1[:, D:].T,
        (bl1 + br1).reshape(1, D), bp1.reshape(1, D), bt1.reshape(1, D))

    s2a, s2b = _make_seg(False)(g1.reshape(2 * N, CH), bsrc, bsid, bcnt,
                                zrows, znc, onesh)
    wr2 = Wr2.T.reshape(R, 2, CH, D)
    _, g2 = _dense_call(
        g1, h1, s2a.reshape(N, R * CH), s2b.reshape(N, R * CH), cnt4,
        Wl2.T, wr2[:, 0].reshape(R * CH, D), wr2[:, 1].reshape(R * CH, D),
        Wp2[:, :D].T, Wp2[:, D:].T, Wt2[:, :D].T, Wt2[:, D:].T,
        (bl2 + br2).reshape(1, D), bp2.reshape(1, D), bt2.reshape(1, D))
    return g2
